# baseline (speedup 1.0000x reference)
import functools

import jax
import jax.numpy as jnp
from jax import lax
from jax.experimental import pallas as pl
from jax.experimental.pallas import tpu as pltpu
from jax.experimental.pallas import tpu_sc as plsc

NUM_EXPERTS = 64
TOPK = 2
LANES = 16


def _sc_gather(flat_ids, table_flat, vocab, num_workers, per_worker):
    mesh = plsc.VectorSubcoreMesh(core_axis_name="c", subcore_axis_name="s")
    nc = 2
    n = num_workers * per_worker

    @functools.partial(
        pl.kernel,
        mesh=mesh,
        out_type=jax.ShapeDtypeStruct((n,), jnp.int32),
        compiler_params=pltpu.CompilerParams(use_tc_tiling_on_sc=False),
        scratch_types=[
            pltpu.VMEM((per_worker,), jnp.int32),
            pltpu.VMEM((per_worker,), jnp.int32),
            pltpu.VMEM((per_worker,), jnp.int32),
            pltpu.VMEM((per_worker,), jnp.int32),
            pltpu.SemaphoreType.DMA,
        ],
    )
    def gather_kernel(tok_hbm, table_hbm, out_hbm, idx0_v, idx1_v, e0_v, e1_v, sem):
        wid = lax.axis_index("s") * nc + lax.axis_index("c")
        base = wid * per_worker
        pltpu.sync_copy(tok_hbm.at[pl.ds(base, per_worker)], idx0_v)

        @pl.loop(0, per_worker, step=LANES)
        def _build(off):
            sl = pl.ds(off, LANES)
            idx1_v[sl] = idx0_v[sl] + vocab

        c0 = pltpu.async_copy(table_hbm.at[idx0_v], e0_v, sem)
        c1 = pltpu.async_copy(table_hbm.at[idx1_v], e1_v, sem)
        c0.wait()
        c1.wait()

        @pl.loop(0, per_worker, step=LANES)
        def _pack(off):
            sl = pl.ds(off, LANES)
            e0_v[sl] = lax.bitwise_or(e0_v[sl], lax.shift_left(e1_v[sl], 8))

        pltpu.sync_copy(e0_v, out_hbm.at[pl.ds(base, per_worker)])

    return gather_kernel(flat_ids, table_flat)


def _tc_expand(codes, n, block_tokens):
    def body(code_ref, probs_ref, map_ref):
        bc = jnp.broadcast_to(code_ref[...], (NUM_EXPERTS, block_tokens))
        iota = lax.broadcasted_iota(jnp.int32, (NUM_EXPERTS, block_tokens), 0)
        m = (iota == (bc & 0xFF)) | (iota == (bc >> 8))
        map_ref[...] = m.astype(jnp.int8)
        probs_ref[...] = jnp.where(m, jnp.float32(1.0 / TOPK), jnp.float32(0.0))

    return pl.pallas_call(
        body,
        grid=(n // block_tokens,),
        in_specs=[pl.BlockSpec((1, block_tokens), lambda i: (0, i))],
        out_specs=[
            pl.BlockSpec((NUM_EXPERTS, block_tokens), lambda i: (0, i)),
            pl.BlockSpec((NUM_EXPERTS, block_tokens), lambda i: (0, i)),
        ],
        out_shape=[
            jax.ShapeDtypeStruct((NUM_EXPERTS, n), jnp.float32),
            jax.ShapeDtypeStruct((NUM_EXPERTS, n), jnp.int8),
        ],
    )(codes)


def kernel(token_ids, tid2eid):
    n = token_ids.size
    num_workers = 32
    per_worker = n // num_workers
    flat_ids = token_ids.reshape(n)
    vocab = tid2eid.shape[0]
    table_flat = tid2eid.T.reshape(2 * vocab)
    codes = _sc_gather(flat_ids, table_flat, vocab, num_workers, per_worker)
    probs_t, map_t = _tc_expand(codes.reshape(1, n), n, block_tokens=4096)
    return probs_t.T, map_t.T.view(jnp.bool_)


# speedup vs baseline: 1.0038x; 1.0038x over previous
import functools

import jax
import jax.numpy as jnp
from jax import lax
from jax.experimental import pallas as pl
from jax.experimental.pallas import tpu as pltpu
from jax.experimental.pallas import tpu_sc as plsc

NUM_EXPERTS = 64
TOPK = 2
LANES = 16


def _sc_gather(flat_ids, table_flat, vocab, num_workers, per_worker):
    mesh = plsc.VectorSubcoreMesh(core_axis_name="c", subcore_axis_name="s")
    nc = 2
    n = num_workers * per_worker

    @functools.partial(
        pl.kernel,
        mesh=mesh,
        out_type=jax.ShapeDtypeStruct((n,), jnp.int32),
        compiler_params=pltpu.CompilerParams(use_tc_tiling_on_sc=False),
        scratch_types=[
            pltpu.VMEM((per_worker,), jnp.int32),
            pltpu.VMEM((per_worker,), jnp.int32),
            pltpu.VMEM((per_worker,), jnp.int32),
            pltpu.VMEM((per_worker,), jnp.int32),
            pltpu.SemaphoreType.DMA,
        ],
    )
    def gather_kernel(tok_hbm, table_hbm, out_hbm, idx0_v, idx1_v, e0_v, e1_v, sem):
        wid = lax.axis_index("s") * nc + lax.axis_index("c")
        base = wid * per_worker
        pltpu.sync_copy(tok_hbm.at[pl.ds(base, per_worker)], idx0_v)

        @pl.loop(0, per_worker, step=LANES)
        def _build(off):
            sl = pl.ds(off, LANES)
            idx1_v[sl] = idx0_v[sl] + vocab

        c0 = pltpu.async_copy(table_hbm.at[idx0_v], e0_v, sem)
        c1 = pltpu.async_copy(table_hbm.at[idx1_v], e1_v, sem)
        c0.wait()
        c1.wait()

        @pl.loop(0, per_worker, step=LANES)
        def _pack(off):
            sl = pl.ds(off, LANES)
            e0_v[sl] = lax.bitwise_or(e0_v[sl], lax.shift_left(e1_v[sl], 8))

        pltpu.sync_copy(e0_v, out_hbm.at[pl.ds(base, per_worker)])

    return gather_kernel(flat_ids, table_flat)


def _tc_expand(codes, n, block_tokens):
    def body(code_ref, probs_ref, map_ref):
        bc = jnp.broadcast_to(code_ref[...], (NUM_EXPERTS, block_tokens))
        iota = lax.broadcasted_iota(jnp.int32, (NUM_EXPERTS, block_tokens), 0)
        m = (iota == (bc & 0xFF)) | (iota == (bc >> 8))
        map_ref[...] = m.astype(jnp.int8)
        probs_ref[...] = jnp.where(m, jnp.float32(1.0 / TOPK), jnp.float32(0.0))

    return pl.pallas_call(
        body,
        grid=(n // block_tokens,),
        in_specs=[pl.BlockSpec((1, block_tokens), lambda i: (0, i))],
        out_specs=[
            pl.BlockSpec((NUM_EXPERTS, block_tokens), lambda i: (0, i)),
            pl.BlockSpec((NUM_EXPERTS, block_tokens), lambda i: (0, i)),
        ],
        out_shape=[
            jax.ShapeDtypeStruct((NUM_EXPERTS, n), jnp.float32),
            jax.ShapeDtypeStruct((NUM_EXPERTS, n), jnp.int8),
        ],
    )(codes)


def kernel(token_ids, tid2eid):
    n = token_ids.size
    num_workers = 32
    per_worker = n // num_workers
    flat_ids = token_ids.reshape(n)
    vocab = tid2eid.shape[0]
    table_flat = tid2eid.T.reshape(2 * vocab)
    codes = _sc_gather(flat_ids, table_flat, vocab, num_workers, per_worker)
    probs_t, map_t = _tc_expand(codes.reshape(1, n), n, block_tokens=32768)
    return probs_t.T, map_t.T.view(jnp.bool_)


# speedup vs baseline: 1.0346x; 1.0307x over previous
import functools

import jax
import jax.numpy as jnp
from jax import lax
from jax.experimental import pallas as pl
from jax.experimental.pallas import tpu as pltpu
from jax.experimental.pallas import tpu_sc as plsc

NUM_EXPERTS = 64
TOPK = 2
LANES = 16


def _sc_gather(flat_ids, table_flat, vocab, num_workers, per_worker):
    mesh = plsc.VectorSubcoreMesh(core_axis_name="c", subcore_axis_name="s")
    nc = 2
    n = num_workers * per_worker

    @functools.partial(
        pl.kernel,
        mesh=mesh,
        out_type=jax.ShapeDtypeStruct((n,), jnp.int32),
        compiler_params=pltpu.CompilerParams(use_tc_tiling_on_sc=False),
        scratch_types=[
            pltpu.VMEM((per_worker,), jnp.int32),
            pltpu.VMEM((per_worker,), jnp.int32),
            pltpu.VMEM((per_worker,), jnp.int32),
            pltpu.VMEM((per_worker,), jnp.int32),
            pltpu.SemaphoreType.DMA,
        ],
    )
    def gather_kernel(tok_hbm, table_hbm, out_hbm, idx0_v, idx1_v, e0_v, e1_v, sem):
        wid = lax.axis_index("s") * nc + lax.axis_index("c")
        base = wid * per_worker
        pltpu.sync_copy(tok_hbm.at[pl.ds(base, per_worker)], idx0_v)

        @pl.loop(0, per_worker, step=LANES)
        def _build(off):
            sl = pl.ds(off, LANES)
            idx1_v[sl] = idx0_v[sl] + vocab

        c0 = pltpu.async_copy(table_hbm.at[idx0_v], e0_v, sem)
        c1 = pltpu.async_copy(table_hbm.at[idx1_v], e1_v, sem)
        c0.wait()
        c1.wait()

        @pl.loop(0, per_worker, step=LANES)
        def _pack(off):
            sl = pl.ds(off, LANES)
            e0_v[sl] = lax.bitwise_or(e0_v[sl], lax.shift_left(e1_v[sl], 8))

        pltpu.sync_copy(e0_v, out_hbm.at[pl.ds(base, per_worker)])

    return gather_kernel(flat_ids, table_flat)


def _tc_expand(codes, n, block_tokens):
    def body(code_ref, probs_ref, map_ref):
        bc = jnp.broadcast_to(code_ref[...], (NUM_EXPERTS, block_tokens))
        iota = lax.broadcasted_iota(jnp.int32, (NUM_EXPERTS, block_tokens), 0)
        m = (iota == (bc & 0xFF)) | (iota == (bc >> 8))
        map_ref[...] = m.astype(jnp.int8)
        probs_ref[...] = jnp.where(m, jnp.float32(1.0 / TOPK), jnp.float32(0.0))

    return pl.pallas_call(
        body,
        grid=(n // block_tokens,),
        in_specs=[pl.BlockSpec((1, block_tokens), lambda i: (0, i))],
        out_specs=[
            pl.BlockSpec((NUM_EXPERTS, block_tokens), lambda i: (0, i)),
            pl.BlockSpec((NUM_EXPERTS, block_tokens), lambda i: (0, i)),
        ],
        out_shape=[
            jax.ShapeDtypeStruct((NUM_EXPERTS, n), jnp.float32),
            jax.ShapeDtypeStruct((NUM_EXPERTS, n), jnp.int8),
        ],
    )(codes)


def kernel(token_ids, tid2eid):
    n = token_ids.size
    num_workers = 32
    per_worker = n // num_workers
    flat_ids = token_ids.reshape(n)
    vocab = tid2eid.shape[0]
    table_flat = tid2eid.T.reshape(2 * vocab)
    codes = _sc_gather(flat_ids, table_flat, vocab, num_workers, per_worker)
    probs_t, map_t = _tc_expand(codes.reshape(1, n), n, block_tokens=16384)
    return probs_t.T, map_t.T.view(jnp.bool_)
